# baseline (speedup 1.0000x reference)
import functools

import jax
import jax.numpy as jnp
from jax import lax
from jax.experimental import pallas as pl
from jax.experimental.pallas import tpu as pltpu
from jax.experimental.pallas import tpu_sc as plsc

N = 10000
D_FEAT = 128
HALF = 64
E = 320000

NC = 2
NS = 16
CHUNK = 128
SUP = 8
E_TILE = 20480
E_PAD = NS * E_TILE
DEG_CHUNKS = E_PAD // (NC * NS * CHUNK)
ROWS = 10240
DUMMY = N
SEG = ROWS // NS
SUB = 80

_mesh = plsc.VectorSubcoreMesh(core_axis_name="c", subcore_axis_name="s")
_sc_params = pltpu.CompilerParams(use_tc_tiling_on_sc=False,
                                 needs_layout_passes=False)


def _deg_body(src2d, ones_hbm, zeros_hbm, deg_out, deg_sh, idx_v, ones_v):
    c = lax.axis_index("c")
    s = lax.axis_index("s")
    wid = c * NS + s
    pltpu.sync_copy(zeros_hbm.at[pl.ds(s * SEG, SEG)],
                    deg_sh.at[pl.ds(s * SEG, SEG)])
    pltpu.sync_copy(ones_hbm, ones_v)
    plsc.subcore_barrier()

    base = wid * DEG_CHUNKS

    def step(j, carry):
        pltpu.sync_copy(src2d.at[pl.ds(base + j * SUP, SUP)], idx_v)
        for k in range(SUP):
            pltpu.sync_copy(ones_v, deg_sh.at[idx_v.at[k]], add=True)
        return carry

    lax.fori_loop(0, DEG_CHUNKS // SUP, step, 0)
    plsc.subcore_barrier()
    pltpu.sync_copy(deg_sh.at[pl.ds(s * SEG, SEG)],
                    deg_out.at[pl.ds(c * ROWS + s * SEG, SEG)])


_deg_kernel = functools.partial(
    pl.kernel,
    out_type=jax.ShapeDtypeStruct((NC * ROWS, 8), jnp.float32),
    mesh=_mesh,
    scratch_types=[
        pltpu.VMEM_SHARED((ROWS, 8), jnp.float32),
        pltpu.VMEM((SUP, CHUNK), jnp.int32),
        pltpu.VMEM((CHUNK, 8), jnp.float32),
    ],
    compiler_params=_sc_params,
)(_deg_body)


def _norm_body(x_ref, xs_ref):
    x = x_ref[...]
    mean = jnp.mean(x, axis=0, keepdims=True)
    xc = x - mean
    var = jnp.sum(xc * xc, axis=0, keepdims=True) * (1.0 / (N - 1))
    std = jnp.sqrt(var)
    std = jnp.where(std == 0.0, 1.0, std)
    xn = xc / std
    xs_ref[0:N, :] = xn[:, 0:HALF]
    xs_ref[N:ROWS, :] = jnp.zeros((ROWS - N, HALF), jnp.float32)
    xs_ref[ROWS:ROWS + N, :] = xn[:, HALF:D_FEAT]
    xs_ref[ROWS + N:2 * ROWS, :] = jnp.zeros((ROWS - N, HALF), jnp.float32)


_norm_kernel = pl.pallas_call(
    _norm_body,
    out_shape=jax.ShapeDtypeStruct((NC * ROWS, HALF), jnp.float32),
)


SCHUNK = 256
NSTEP = E_TILE // SCHUNK
SPL = 11


def _rsqrt16(a):
    i = plsc.bitcast(a, jnp.int32)
    i = 0x5F3759DF - lax.shift_right_logical(i, 1)
    y = plsc.bitcast(i, jnp.float32)
    for _ in range(3):
        y = y * (1.5 - 0.5 * a * y * y)
    return y


def _main_body(xn_hbm, src1d, dst1d, dsth1d, degp_hbm, zeros_hbm,
               out_hbm, xsh_out,
               agg_sh, xs_sh, d0, d1, s0, s1, rows_v, degb, dinv_v,
               gsem, ssem, isem):
    c = lax.axis_index("c")
    s = lax.axis_index("s")
    base = s * SEG
    iota16 = lax.broadcasted_iota(jnp.int32, (16,), 0)

    pltpu.sync_copy(zeros_hbm.at[pl.ds(base, SEG)],
                    agg_sh.at[pl.ds(base, SEG)])

    def pro(t, carry):
        r0 = base + t * SUB

        @pl.when(r0 < N)
        def _():
            pltpu.sync_copy(xn_hbm.at[pl.ds(c * ROWS + r0, SUB)],
                            rows_v.at[pl.ds(0, SUB)])
            pltpu.sync_copy(degp_hbm.at[pl.ds(r0, SUB)],
                            degb.at[pl.ds(0, SUB)])
            pltpu.sync_copy(degp_hbm.at[pl.ds(ROWS + r0, SUB)],
                            degb.at[pl.ds(SUB, SUB)])
            for g in range(SUB // 16):
                ridx = iota16 + g * 16
                z16 = jnp.zeros((16,), jnp.int32)
                dg0 = plsc.load_gather(degb, [ridx, z16])
                dg1 = plsc.load_gather(degb, [ridx + SUB, z16])
                y = _rsqrt16(dg0 + dg1 + 2.0)
                dinv_v[pl.ds(t * SUB + g * 16, 16)] = y
                for rr in range(16):
                    r = g * 16 + rr
                    dsp = jnp.full((16,), y[rr], jnp.float32)
                    for cg in range(HALF // 16):
                        col = pl.ds(cg * 16, 16)
                        rows_v[r, col] = rows_v[r, col] * dsp
            pltpu.sync_copy(rows_v.at[pl.ds(0, SUB)],
                            xs_sh.at[pl.ds(r0, SUB)])
            pltpu.sync_copy(rows_v.at[pl.ds(0, SUB)],
                            xsh_out.at[pl.ds(c * ROWS + r0, SUB)])
        return carry

    lax.fori_loop(0, SEG // SUB, pro, 0)

    dbank = [d0, d1]
    sbank = [s0, s1]

    def load_idx(j, bank):
        off = s * E_TILE + j * SCHUNK

        @pl.when(s < SPL)
        def _():
            pltpu.async_copy(dst1d.at[pl.ds(off, SCHUNK)],
                             dbank[bank], isem.at[bank])

        @pl.when(s >= SPL)
        def _():
            pltpu.async_copy(dsth1d.at[pl.ds(c * E_PAD + off, SCHUNK)],
                             dbank[bank], isem.at[bank])

        pltpu.async_copy(src1d.at[pl.ds(off, SCHUNK)],
                         sbank[bank], isem.at[bank])

    def wait_idx(bank):
        for _ in range(2):
            pltpu.make_async_copy(src1d.at[pl.ds(0, SCHUNK)],
                                  sbank[bank], isem.at[bank]).wait()

    def fire_g(bank):
        @pl.when(s < SPL)
        def _():
            pltpu.async_copy(xs_sh.at[dbank[bank]],
                             rows_v.at[pl.ds(bank * SCHUNK, SCHUNK)],
                             gsem.at[bank])

        @pl.when(s >= SPL)
        def _():
            pltpu.async_copy(xsh_out.at[dbank[bank]],
                             rows_v.at[pl.ds(bank * SCHUNK, SCHUNK)],
                             gsem.at[bank])

    def drain_g(bank):
        pltpu.make_async_copy(xs_sh.at[dbank[bank]],
                              rows_v.at[pl.ds(bank * SCHUNK, SCHUNK)],
                              gsem.at[bank]).wait()

    def fire_s(bank):
        pltpu.async_copy(rows_v.at[pl.ds(bank * SCHUNK, SCHUNK)],
                         agg_sh.at[sbank[bank]], ssem.at[bank], add=True)

    def drain_s(bank):
        pltpu.make_async_copy(rows_v.at[pl.ds(bank * SCHUNK, SCHUNK)],
                              agg_sh.at[sbank[bank]], ssem.at[bank]).wait()

    load_idx(0, 0)
    load_idx(1, 1)
    plsc.subcore_barrier()
    wait_idx(0)
    fire_g(0)
    wait_idx(1)
    fire_g(1)

    def body(i, carry):
        drain_g(0)
        fire_s(0)
        drain_g(1)
        drain_s(0)
        load_idx(2 * i + 2, 0)
        fire_s(1)
        wait_idx(0)
        fire_g(0)
        drain_s(1)
        load_idx(2 * i + 3, 1)
        wait_idx(1)
        fire_g(1)
        return carry

    lax.fori_loop(0, NSTEP // 2 - 1, body, 0)
    drain_g(0)
    fire_s(0)
    drain_g(1)
    fire_s(1)
    drain_s(0)
    drain_s(1)

    plsc.subcore_barrier()

    def epi(t, carry):
        r0 = base + t * SUB

        @pl.when(r0 < N)
        def _():
            pltpu.sync_copy(agg_sh.at[pl.ds(r0, SUB)],
                            rows_v.at[pl.ds(0, SUB)])
            pltpu.sync_copy(xs_sh.at[pl.ds(r0, SUB)],
                            rows_v.at[pl.ds(SUB, SUB)])
            for g in range(SUB // 16):
                yv = dinv_v[pl.ds(t * SUB + g * 16, 16)]
                for rr in range(16):
                    r = g * 16 + rr
                    dsp = jnp.full((16,), yv[rr], jnp.float32)
                    for cg in range(HALF // 16):
                        col = pl.ds(cg * 16, 16)
                        rows_v[2 * SUB + r, col] = (
                            rows_v[r, col] + 2.0 * rows_v[SUB + r, col]) * dsp
            pltpu.sync_copy(rows_v.at[pl.ds(2 * SUB, SUB)],
                            out_hbm.at[pl.ds(r0, SUB), pl.ds(c * HALF, HALF)])
        return carry

    lax.fori_loop(0, SEG // SUB, epi, 0)


_main_kernel = functools.partial(
    pl.kernel,
    out_type=(jax.ShapeDtypeStruct((N, D_FEAT), jnp.float32),
              jax.ShapeDtypeStruct((NC * ROWS, HALF), jnp.float32)),
    mesh=_mesh,
    scratch_types=[
        pltpu.VMEM_SHARED((ROWS, HALF), jnp.float32),
        pltpu.VMEM_SHARED((ROWS, HALF), jnp.float32),
        pltpu.VMEM((SCHUNK,), jnp.int32),
        pltpu.VMEM((SCHUNK,), jnp.int32),
        pltpu.VMEM((SCHUNK,), jnp.int32),
        pltpu.VMEM((SCHUNK,), jnp.int32),
        pltpu.VMEM((2 * SCHUNK, HALF), jnp.float32),
        pltpu.VMEM((2 * SUB, 8), jnp.float32),
        pltpu.VMEM((SEG,), jnp.float32),
        pltpu.SemaphoreType.DMA((2,)),
        pltpu.SemaphoreType.DMA((2,)),
        pltpu.SemaphoreType.DMA((2,)),
    ],
    compiler_params=_sc_params,
)(_main_body)


def kernel(x, edge_index):
    src = edge_index[0].astype(jnp.int32)
    dst = edge_index[1].astype(jnp.int32)
    pad = E_PAD - E
    src_p = jnp.concatenate([src, jnp.full((pad,), DUMMY, jnp.int32)])
    dst_p = jnp.concatenate([dst, jnp.zeros((pad,), jnp.int32)])
    src2d = src_p.reshape(-1, CHUNK)

    ones8 = jnp.ones((CHUNK, 8), jnp.float32)
    zeros8 = jnp.zeros((ROWS, 8), jnp.float32)
    zeros64 = jnp.zeros((ROWS, HALF), jnp.float32)

    dsth = jnp.concatenate([dst_p, dst_p + ROWS])
    degp = _deg_kernel(src2d, ones8, zeros8)
    xn_cat = _norm_kernel(x)
    out, _ = _main_kernel(xn_cat, src_p, dst_p, dsth, degp, zeros64)
    return out


# speedup vs baseline: 1.8387x; 1.8387x over previous
import functools

import jax
import jax.numpy as jnp
from jax import lax
from jax.experimental import pallas as pl
from jax.experimental.pallas import tpu as pltpu
from jax.experimental.pallas import tpu_sc as plsc

N = 10000
D_FEAT = 128
HALF = 64
E = 320000

NC = 2
NS = 16
CHUNK = 128
SUP = 8
E_TILE = 20480
E_PAD = NS * E_TILE
DEG_CHUNKS = E_PAD // (NC * NS * CHUNK)
ROWS = 10240
DUMMY = N
SEG = ROWS // NS
SUB = 80

_mesh = plsc.VectorSubcoreMesh(core_axis_name="c", subcore_axis_name="s")
_sc_params = pltpu.CompilerParams(use_tc_tiling_on_sc=False,
                                 needs_layout_passes=False)


DEGC = 1024
DEG_E = E_PAD // (NC * NS)


def _deg_body(src1d, ones_hbm, zeros_hbm, deg_out, deg_sh, i0, i1, ones_v,
              isem):
    c = lax.axis_index("c")
    s = lax.axis_index("s")
    wid = c * NS + s
    pltpu.sync_copy(zeros_hbm.at[pl.ds(s * SEG, SEG)],
                    deg_sh.at[pl.ds(s * SEG, SEG)])
    pltpu.sync_copy(ones_hbm, ones_v)
    base = wid * DEG_E
    ibank = [i0, i1]

    def load_idx(j, bank):
        pltpu.async_copy(src1d.at[pl.ds(base + j * DEGC, DEGC)],
                         ibank[bank], isem.at[bank])

    def wait_idx(bank):
        pltpu.make_async_copy(src1d.at[pl.ds(0, DEGC)],
                              ibank[bank], isem.at[bank]).wait()

    load_idx(0, 0)
    load_idx(1, 1)
    plsc.subcore_barrier()

    def step(i, carry):
        j0 = 2 * i
        wait_idx(0)
        pltpu.sync_copy(ones_v, deg_sh.at[i0], add=True)
        load_idx(j0 + 2, 0)
        wait_idx(1)
        pltpu.sync_copy(ones_v, deg_sh.at[i1], add=True)
        load_idx(j0 + 3, 1)
        return carry

    lax.fori_loop(0, DEG_E // DEGC // 2 - 1, step, 0)
    wait_idx(0)
    pltpu.sync_copy(ones_v, deg_sh.at[i0], add=True)
    wait_idx(1)
    pltpu.sync_copy(ones_v, deg_sh.at[i1], add=True)
    plsc.subcore_barrier()
    pltpu.sync_copy(deg_sh.at[pl.ds(s * SEG, SEG)],
                    deg_out.at[pl.ds(c * ROWS + s * SEG, SEG)])


_deg_kernel = functools.partial(
    pl.kernel,
    out_type=jax.ShapeDtypeStruct((NC * ROWS, 8), jnp.float32),
    mesh=_mesh,
    scratch_types=[
        pltpu.VMEM_SHARED((ROWS, 8), jnp.float32),
        pltpu.VMEM((DEGC,), jnp.int32),
        pltpu.VMEM((DEGC,), jnp.int32),
        pltpu.VMEM((DEGC, 8), jnp.float32),
        pltpu.SemaphoreType.DMA((2,)),
    ],
    compiler_params=_sc_params,
)(_deg_body)


def _norm_body(x_ref, xs_ref):
    x = x_ref[...]
    mean = jnp.mean(x, axis=0, keepdims=True)
    xc = x - mean
    var = jnp.sum(xc * xc, axis=0, keepdims=True) * (1.0 / (N - 1))
    std = jnp.sqrt(var)
    std = jnp.where(std == 0.0, 1.0, std)
    xn = xc / std
    xs_ref[0:N, :] = xn[:, 0:HALF]
    xs_ref[N:ROWS, :] = jnp.zeros((ROWS - N, HALF), jnp.float32)
    xs_ref[ROWS:ROWS + N, :] = xn[:, HALF:D_FEAT]
    xs_ref[ROWS + N:2 * ROWS, :] = jnp.zeros((ROWS - N, HALF), jnp.float32)


_norm_kernel = pl.pallas_call(
    _norm_body,
    out_shape=jax.ShapeDtypeStruct((NC * ROWS, HALF), jnp.float32),
)


SCHUNK = 256
NSTEP = E_TILE // SCHUNK


def _rsqrt16(a):
    i = plsc.bitcast(a, jnp.int32)
    i = 0x5F3759DF - lax.shift_right_logical(i, 1)
    y = plsc.bitcast(i, jnp.float32)
    for _ in range(3):
        y = y * (1.5 - 0.5 * a * y * y)
    return y


def _main_body(xn_hbm, src1d, dst1d, degp_hbm, zeros_hbm, out_hbm,
               agg_sh, xs_sh, d0, d1, s0, s1, rows_v, degb, dinv_v,
               gsem, ssem, isem):
    c = lax.axis_index("c")
    s = lax.axis_index("s")
    base = s * SEG
    iota16 = lax.broadcasted_iota(jnp.int32, (16,), 0)

    pltpu.sync_copy(zeros_hbm.at[pl.ds(base, SEG)],
                    agg_sh.at[pl.ds(base, SEG)])

    def pro(t, carry):
        r0 = base + t * SUB

        @pl.when(r0 < N)
        def _():
            pltpu.sync_copy(xn_hbm.at[pl.ds(c * ROWS + r0, SUB)],
                            rows_v.at[pl.ds(0, SUB)])
            pltpu.sync_copy(degp_hbm.at[pl.ds(r0, SUB)],
                            degb.at[pl.ds(0, SUB)])
            pltpu.sync_copy(degp_hbm.at[pl.ds(ROWS + r0, SUB)],
                            degb.at[pl.ds(SUB, SUB)])
            for g in range(SUB // 16):
                ridx = iota16 + g * 16
                z16 = jnp.zeros((16,), jnp.int32)
                dg0 = plsc.load_gather(degb, [ridx, z16])
                dg1 = plsc.load_gather(degb, [ridx + SUB, z16])
                y = _rsqrt16(dg0 + dg1 + 2.0)
                dinv_v[pl.ds(t * SUB + g * 16, 16)] = y
                for rr in range(16):
                    r = g * 16 + rr
                    dsp = jnp.full((16,), y[rr], jnp.float32)
                    for cg in range(HALF // 16):
                        col = pl.ds(cg * 16, 16)
                        rows_v[r, col] = rows_v[r, col] * dsp
            pltpu.sync_copy(rows_v.at[pl.ds(0, SUB)],
                            xs_sh.at[pl.ds(r0, SUB)])
        return carry

    lax.fori_loop(0, SEG // SUB, pro, 0)

    dbank = [d0, d1]
    sbank = [s0, s1]

    def load_idx(j, bank):
        off = s * E_TILE + j * SCHUNK
        pltpu.async_copy(dst1d.at[pl.ds(off, SCHUNK)],
                         dbank[bank], isem.at[bank])
        pltpu.async_copy(src1d.at[pl.ds(off, SCHUNK)],
                         sbank[bank], isem.at[bank])

    def wait_idx(bank):
        for _ in range(2):
            pltpu.make_async_copy(src1d.at[pl.ds(0, SCHUNK)],
                                  sbank[bank], isem.at[bank]).wait()

    def fire_g(bank):
        pltpu.async_copy(xs_sh.at[dbank[bank]],
                         rows_v.at[pl.ds(bank * SCHUNK, SCHUNK)],
                         gsem.at[bank])

    def drain_g(bank):
        pltpu.make_async_copy(xs_sh.at[dbank[bank]],
                              rows_v.at[pl.ds(bank * SCHUNK, SCHUNK)],
                              gsem.at[bank]).wait()

    def fire_s(bank):
        pltpu.async_copy(rows_v.at[pl.ds(bank * SCHUNK, SCHUNK)],
                         agg_sh.at[sbank[bank]], ssem.at[bank], add=True)

    def drain_s(bank):
        pltpu.make_async_copy(rows_v.at[pl.ds(bank * SCHUNK, SCHUNK)],
                              agg_sh.at[sbank[bank]], ssem.at[bank]).wait()

    load_idx(0, 0)
    load_idx(1, 1)
    plsc.subcore_barrier()
    wait_idx(0)
    fire_g(0)
    wait_idx(1)
    fire_g(1)

    def body(i, carry):
        drain_g(0)
        fire_s(0)
        drain_g(1)
        drain_s(0)
        load_idx(2 * i + 2, 0)
        fire_s(1)
        wait_idx(0)
        fire_g(0)
        drain_s(1)
        load_idx(2 * i + 3, 1)
        wait_idx(1)
        fire_g(1)
        return carry

    lax.fori_loop(0, NSTEP // 2 - 1, body, 0)
    drain_g(0)
    fire_s(0)
    drain_g(1)
    fire_s(1)
    drain_s(0)
    drain_s(1)

    plsc.subcore_barrier()

    def epi(t, carry):
        r0 = base + t * SUB

        @pl.when(r0 < N)
        def _():
            pltpu.sync_copy(agg_sh.at[pl.ds(r0, SUB)],
                            rows_v.at[pl.ds(0, SUB)])
            pltpu.sync_copy(xs_sh.at[pl.ds(r0, SUB)],
                            rows_v.at[pl.ds(SUB, SUB)])
            for g in range(SUB // 16):
                yv = dinv_v[pl.ds(t * SUB + g * 16, 16)]
                for rr in range(16):
                    r = g * 16 + rr
                    dsp = jnp.full((16,), yv[rr], jnp.float32)
                    for cg in range(HALF // 16):
                        col = pl.ds(cg * 16, 16)
                        rows_v[2 * SUB + r, col] = (
                            rows_v[r, col] + 2.0 * rows_v[SUB + r, col]) * dsp
            pltpu.sync_copy(rows_v.at[pl.ds(2 * SUB, SUB)],
                            out_hbm.at[pl.ds(r0, SUB), pl.ds(c * HALF, HALF)])
        return carry

    lax.fori_loop(0, SEG // SUB, epi, 0)


_main_kernel = functools.partial(
    pl.kernel,
    out_type=jax.ShapeDtypeStruct((N, D_FEAT), jnp.float32),
    mesh=_mesh,
    scratch_types=[
        pltpu.VMEM_SHARED((ROWS, HALF), jnp.float32),
        pltpu.VMEM_SHARED((ROWS, HALF), jnp.float32),
        pltpu.VMEM((SCHUNK,), jnp.int32),
        pltpu.VMEM((SCHUNK,), jnp.int32),
        pltpu.VMEM((SCHUNK,), jnp.int32),
        pltpu.VMEM((SCHUNK,), jnp.int32),
        pltpu.VMEM((2 * SCHUNK, HALF), jnp.float32),
        pltpu.VMEM((2 * SUB, 8), jnp.float32),
        pltpu.VMEM((SEG,), jnp.float32),
        pltpu.SemaphoreType.DMA((2,)),
        pltpu.SemaphoreType.DMA((2,)),
        pltpu.SemaphoreType.DMA((2,)),
    ],
    compiler_params=_sc_params,
)(_main_body)


def kernel(x, edge_index):
    src = edge_index[0].astype(jnp.int32)
    dst = edge_index[1].astype(jnp.int32)
    pad = E_PAD - E
    src_p = jnp.concatenate([src, jnp.full((pad,), DUMMY, jnp.int32)])
    dst_p = jnp.concatenate([dst, jnp.zeros((pad,), jnp.int32)])

    ones8 = jnp.ones((1024, 8), jnp.float32)
    zeros8 = jnp.zeros((ROWS, 8), jnp.float32)
    zeros64 = jnp.zeros((ROWS, HALF), jnp.float32)

    degp = _deg_kernel(src_p, ones8, zeros8)
    xn_cat = _norm_kernel(x)
    return _main_kernel(xn_cat, src_p, dst_p, degp, zeros64)


# speedup vs baseline: 1.8913x; 1.0286x over previous
import functools

import jax
import jax.numpy as jnp
from jax import lax
from jax.experimental import pallas as pl
from jax.experimental.pallas import tpu as pltpu
from jax.experimental.pallas import tpu_sc as plsc

N = 10000
D_FEAT = 128
HALF = 64
E = 320000

NC = 2
NS = 16
CHUNK = 128
SUP = 8
E_TILE = 20480
E_PAD = NS * E_TILE
DEG_CHUNKS = E_PAD // (NC * NS * CHUNK)
ROWS = 10240
DUMMY = N
SEG = ROWS // NS
SUB = 80

_mesh = plsc.VectorSubcoreMesh(core_axis_name="c", subcore_axis_name="s")
_sc_params = pltpu.CompilerParams(use_tc_tiling_on_sc=False,
                                 needs_layout_passes=False)


DEGC = 1024
DEG_E = E_PAD // (NC * NS)


def _deg_body(src1d, ones_hbm, zeros_hbm, deg_out, deg_sh, i0, i1, ones_v,
              isem):
    c = lax.axis_index("c")
    s = lax.axis_index("s")
    wid = c * NS + s
    pltpu.sync_copy(zeros_hbm.at[pl.ds(s * SEG, SEG)],
                    deg_sh.at[pl.ds(s * SEG, SEG)])
    pltpu.sync_copy(ones_hbm, ones_v)
    base = wid * DEG_E
    ibank = [i0, i1]

    def load_idx(j, bank):
        pltpu.async_copy(src1d.at[pl.ds(base + j * DEGC, DEGC)],
                         ibank[bank], isem.at[bank])

    def wait_idx(bank):
        pltpu.make_async_copy(src1d.at[pl.ds(0, DEGC)],
                              ibank[bank], isem.at[bank]).wait()

    load_idx(0, 0)
    load_idx(1, 1)
    plsc.subcore_barrier()

    def step(i, carry):
        j0 = 2 * i
        wait_idx(0)
        pltpu.sync_copy(ones_v, deg_sh.at[i0], add=True)
        load_idx(j0 + 2, 0)
        wait_idx(1)
        pltpu.sync_copy(ones_v, deg_sh.at[i1], add=True)
        load_idx(j0 + 3, 1)
        return carry

    lax.fori_loop(0, DEG_E // DEGC // 2 - 1, step, 0)
    wait_idx(0)
    pltpu.sync_copy(ones_v, deg_sh.at[i0], add=True)
    wait_idx(1)
    pltpu.sync_copy(ones_v, deg_sh.at[i1], add=True)
    plsc.subcore_barrier()
    pltpu.sync_copy(deg_sh.at[pl.ds(s * SEG, SEG)],
                    deg_out.at[pl.ds(c * ROWS + s * SEG, SEG)])


_deg_kernel = functools.partial(
    pl.kernel,
    out_type=jax.ShapeDtypeStruct((NC * ROWS, 8), jnp.float32),
    mesh=_mesh,
    scratch_types=[
        pltpu.VMEM_SHARED((ROWS, 8), jnp.float32),
        pltpu.VMEM((DEGC,), jnp.int32),
        pltpu.VMEM((DEGC,), jnp.int32),
        pltpu.VMEM((DEGC, 8), jnp.float32),
        pltpu.SemaphoreType.DMA((2,)),
    ],
    compiler_params=_sc_params,
)(_deg_body)


def _norm_body(x_ref, xs_ref):
    x = x_ref[...]
    mean = jnp.mean(x, axis=0, keepdims=True)
    xc = x - mean
    var = jnp.sum(xc * xc, axis=0, keepdims=True) * (1.0 / (N - 1))
    std = jnp.sqrt(var)
    std = jnp.where(std == 0.0, 1.0, std)
    xn = xc / std
    xs_ref[0:N, :] = xn[:, 0:HALF]
    xs_ref[N:ROWS, :] = jnp.zeros((ROWS - N, HALF), jnp.float32)
    xs_ref[ROWS:ROWS + N, :] = xn[:, HALF:D_FEAT]
    xs_ref[ROWS + N:2 * ROWS, :] = jnp.zeros((ROWS - N, HALF), jnp.float32)


_norm_kernel = pl.pallas_call(
    _norm_body,
    out_shape=jax.ShapeDtypeStruct((NC * ROWS, HALF), jnp.float32),
)


SCHUNK = 320
NSTEP = E_TILE // SCHUNK


def _rsqrt16(a):
    i = plsc.bitcast(a, jnp.int32)
    i = 0x5F3759DF - lax.shift_right_logical(i, 1)
    y = plsc.bitcast(i, jnp.float32)
    for _ in range(3):
        y = y * (1.5 - 0.5 * a * y * y)
    return y


def _main_body(xn_hbm, src1d, dst1d, degp_hbm, zeros_hbm, out_hbm,
               agg_sh, xs_sh, d0, d1, s0, s1, rows_v, degb, dinv_v,
               gsem, ssem, isem):
    c = lax.axis_index("c")
    s = lax.axis_index("s")
    base = s * SEG
    iota16 = lax.broadcasted_iota(jnp.int32, (16,), 0)

    pltpu.sync_copy(zeros_hbm.at[pl.ds(base, SEG)],
                    agg_sh.at[pl.ds(base, SEG)])

    def pro(t, carry):
        r0 = base + t * SUB

        @pl.when(r0 < N)
        def _():
            pltpu.sync_copy(xn_hbm.at[pl.ds(c * ROWS + r0, SUB)],
                            rows_v.at[pl.ds(0, SUB)])
            pltpu.sync_copy(degp_hbm.at[pl.ds(r0, SUB)],
                            degb.at[pl.ds(0, SUB)])
            pltpu.sync_copy(degp_hbm.at[pl.ds(ROWS + r0, SUB)],
                            degb.at[pl.ds(SUB, SUB)])
            for g in range(SUB // 16):
                ridx = iota16 + g * 16
                z16 = jnp.zeros((16,), jnp.int32)
                dg0 = plsc.load_gather(degb, [ridx, z16])
                dg1 = plsc.load_gather(degb, [ridx + SUB, z16])
                y = _rsqrt16(dg0 + dg1 + 2.0)
                dinv_v[pl.ds(t * SUB + g * 16, 16)] = y
                for rr in range(16):
                    r = g * 16 + rr
                    dsp = jnp.full((16,), y[rr], jnp.float32)
                    for cg in range(HALF // 16):
                        col = pl.ds(cg * 16, 16)
                        rows_v[r, col] = rows_v[r, col] * dsp
            pltpu.sync_copy(rows_v.at[pl.ds(0, SUB)],
                            xs_sh.at[pl.ds(r0, SUB)])
        return carry

    lax.fori_loop(0, SEG // SUB, pro, 0)

    dbank = [d0, d1]
    sbank = [s0, s1]

    def load_idx(j, bank):
        off = s * E_TILE + j * SCHUNK
        pltpu.async_copy(dst1d.at[pl.ds(off, SCHUNK)],
                         dbank[bank], isem.at[bank])
        pltpu.async_copy(src1d.at[pl.ds(off, SCHUNK)],
                         sbank[bank], isem.at[bank])

    def wait_idx(bank):
        for _ in range(2):
            pltpu.make_async_copy(src1d.at[pl.ds(0, SCHUNK)],
                                  sbank[bank], isem.at[bank]).wait()

    def fire_g(bank):
        pltpu.async_copy(xs_sh.at[dbank[bank]],
                         rows_v.at[pl.ds(bank * SCHUNK, SCHUNK)],
                         gsem.at[bank])

    def drain_g(bank):
        pltpu.make_async_copy(xs_sh.at[dbank[bank]],
                              rows_v.at[pl.ds(bank * SCHUNK, SCHUNK)],
                              gsem.at[bank]).wait()

    def fire_s(bank):
        pltpu.async_copy(rows_v.at[pl.ds(bank * SCHUNK, SCHUNK)],
                         agg_sh.at[sbank[bank]], ssem.at[bank], add=True)

    def drain_s(bank):
        pltpu.make_async_copy(rows_v.at[pl.ds(bank * SCHUNK, SCHUNK)],
                              agg_sh.at[sbank[bank]], ssem.at[bank]).wait()

    load_idx(0, 0)
    load_idx(1, 1)
    plsc.subcore_barrier()
    wait_idx(0)
    fire_g(0)
    wait_idx(1)
    fire_g(1)

    def body(i, carry):
        drain_g(0)
        fire_s(0)
        drain_g(1)
        drain_s(0)
        load_idx(2 * i + 2, 0)
        fire_s(1)
        wait_idx(0)
        fire_g(0)
        drain_s(1)
        load_idx(2 * i + 3, 1)
        wait_idx(1)
        fire_g(1)
        return carry

    lax.fori_loop(0, NSTEP // 2 - 1, body, 0)
    drain_g(0)
    fire_s(0)
    drain_g(1)
    fire_s(1)
    drain_s(0)
    drain_s(1)

    plsc.subcore_barrier()

    def epi(t, carry):
        r0 = base + t * SUB

        @pl.when(r0 < N)
        def _():
            pltpu.sync_copy(agg_sh.at[pl.ds(r0, SUB)],
                            rows_v.at[pl.ds(0, SUB)])
            pltpu.sync_copy(xs_sh.at[pl.ds(r0, SUB)],
                            rows_v.at[pl.ds(SUB, SUB)])
            for g in range(SUB // 16):
                yv = dinv_v[pl.ds(t * SUB + g * 16, 16)]
                for rr in range(16):
                    r = g * 16 + rr
                    dsp = jnp.full((16,), yv[rr], jnp.float32)
                    for cg in range(HALF // 16):
                        col = pl.ds(cg * 16, 16)
                        rows_v[2 * SUB + r, col] = (
                            rows_v[r, col] + 2.0 * rows_v[SUB + r, col]) * dsp
            pltpu.sync_copy(rows_v.at[pl.ds(2 * SUB, SUB)],
                            out_hbm.at[pl.ds(r0, SUB), pl.ds(c * HALF, HALF)])
        return carry

    lax.fori_loop(0, SEG // SUB, epi, 0)


_main_kernel = functools.partial(
    pl.kernel,
    out_type=jax.ShapeDtypeStruct((N, D_FEAT), jnp.float32),
    mesh=_mesh,
    scratch_types=[
        pltpu.VMEM_SHARED((ROWS, HALF), jnp.float32),
        pltpu.VMEM_SHARED((ROWS, HALF), jnp.float32),
        pltpu.VMEM((SCHUNK,), jnp.int32),
        pltpu.VMEM((SCHUNK,), jnp.int32),
        pltpu.VMEM((SCHUNK,), jnp.int32),
        pltpu.VMEM((SCHUNK,), jnp.int32),
        pltpu.VMEM((2 * SCHUNK, HALF), jnp.float32),
        pltpu.VMEM((2 * SUB, 8), jnp.float32),
        pltpu.VMEM((SEG,), jnp.float32),
        pltpu.SemaphoreType.DMA((2,)),
        pltpu.SemaphoreType.DMA((2,)),
        pltpu.SemaphoreType.DMA((2,)),
    ],
    compiler_params=_sc_params,
)(_main_body)


def kernel(x, edge_index):
    src = edge_index[0].astype(jnp.int32)
    dst = edge_index[1].astype(jnp.int32)
    pad = E_PAD - E
    src_p = jnp.concatenate([src, jnp.full((pad,), DUMMY, jnp.int32)])
    dst_p = jnp.concatenate([dst, jnp.zeros((pad,), jnp.int32)])

    ones8 = jnp.ones((1024, 8), jnp.float32)
    zeros8 = jnp.zeros((ROWS, 8), jnp.float32)
    zeros64 = jnp.zeros((ROWS, HALF), jnp.float32)

    degp = _deg_kernel(src_p, ones8, zeros8)
    xn_cat = _norm_kernel(x)
    return _main_kernel(xn_cat, src_p, dst_p, degp, zeros64)
